# baseline (speedup 1.0000x reference)
import jax
import jax.numpy as jnp
from jax import lax
from jax.experimental import pallas as pl
from jax.experimental.pallas import tpu as pltpu
from jax.experimental.pallas import tpu_sc as plsc

B, L, V, D = 16384, 50, 1000000, 32
NC, NS = 2, 16
NW = NC * NS
RW = B // NW
C = 16
NCHUNK = RW // C
S = C * L


def _body(id_sp_hbm, id_flat_hbm, wu_flat_hbm, wh_hbm, out_hbm,
          ids_v, rows_v, sp_idx_v, spg_v, fm_v, st_v, out_t_v, w0_v,
          sem, sem2):
    wid = lax.axis_index("s") * NC + lax.axis_index("c")
    base = wid * RW

    pltpu.sync_copy(wh_hbm.at[0], w0_v)
    pltpu.sync_copy(id_sp_hbm.at[pl.ds(base, RW)], sp_idx_v)

    for k in range(RW // 16):
        v = sp_idx_v[pl.ds(k * 16, 16)]
        for j in range(D):
            spg_v[pl.ds(j * RW + k * 16, 16)] = v + j * V
    pltpu.async_copy(wu_flat_hbm.at[spg_v], fm_v, sem2)

    pltpu.sync_copy(id_flat_hbm.at[pl.ds(base * L, S)], ids_v.at[0])
    pltpu.async_copy(wh_hbm.at[ids_v.at[0]], rows_v.at[0], sem)

    pltpu.make_async_copy(wu_flat_hbm.at[spg_v], fm_v, sem2).wait()

    w00 = w0_v[pl.ds(0, 16)]
    w01 = w0_v[pl.ds(16, 16)]
    lane = lax.iota(jnp.int32, 16)
    pat32 = lane * 32

    def chunk(g, carry):
        p = lax.rem(g, 2)
        q = 1 - p
        row0 = base + g * C

        @pl.when(g < NCHUNK - 1)
        def _():
            pltpu.sync_copy(
                id_flat_hbm.at[pl.ds((row0 + C) * L, S)], ids_v.at[q])
            pltpu.async_copy(wh_hbm.at[ids_v.at[q]], rows_v.at[q], sem)

        pltpu.make_async_copy(wh_hbm.at[ids_v.at[p]], rows_v.at[p], sem).wait()

        for i in range(C):
            o = i * L
            e0 = ids_v[p, pl.ds(o, 16)]
            e1 = ids_v[p, pl.ds(o + 16, 16)]
            e2 = ids_v[p, pl.ds(o + 32, 16)]
            e3 = ids_v[p, pl.ds(o + 34, 16)]
            zc = (plsc.all_reduce_population_count(e0 == 0)
                  + plsc.all_reduce_population_count(e1 == 0)
                  + plsc.all_reduce_population_count(e2 == 0)
                  + plsc.all_reduce_population_count((e3 == 0) & (lane >= 14))
                  ).astype(jnp.float32)

            a = [rows_v[p, o + k, pl.ds(0, 16)] for k in range(4)]
            b = [rows_v[p, o + k, pl.ds(16, 16)] for k in range(4)]
            for l in range(4, L - 2, 4):
                for k in range(4):
                    a[k] = a[k] + rows_v[p, o + l + k, pl.ds(0, 16)]
                    b[k] = b[k] + rows_v[p, o + l + k, pl.ds(16, 16)]
            a[0] = a[0] + rows_v[p, o + L - 2, pl.ds(0, 16)]
            b[0] = b[0] + rows_v[p, o + L - 2, pl.ds(16, 16)]
            a[1] = a[1] + rows_v[p, o + L - 1, pl.ds(0, 16)]
            b[1] = b[1] + rows_v[p, o + L - 1, pl.ds(16, 16)]
            a0 = (a[0] + a[1]) + (a[2] + a[3])
            a1 = (b[0] + b[1]) + (b[2] + b[3])

            inv = 1.0 / (jnp.float32(L) - zc + 1e-8)
            st_v[pl.ds(i * 32, 16)] = (a0 - zc * w00) * inv
            st_v[pl.ds(i * 32 + 16, 16)] = (a1 - zc * w01) * inv

        for j in range(D):
            out_t_v[j, pl.ds(g * C, C)] = plsc.load_gather(st_v, [pat32 + j])
        return carry

    lax.fori_loop(0, NCHUNK, chunk, 0)
    pltpu.sync_copy(out_t_v, out_hbm.at[1, :, pl.ds(base, RW)])
    descs = [
        pltpu.async_copy(fm_v.at[pl.ds(j * RW, RW)],
                         out_hbm.at[0, j, pl.ds(base, RW)], sem2)
        for j in range(D)
    ]
    for d in descs:
        d.wait()


@jax.jit
def kernel(id_sparse, id_seq, W_user, W_hist):
    id_flat = id_seq.reshape(B * L)
    wu_flat = lax.optimization_barrier(W_user.T).reshape(D * V)
    mesh = plsc.VectorSubcoreMesh(core_axis_name="c", subcore_axis_name="s")
    run = pl.kernel(
        _body,
        out_type=jax.ShapeDtypeStruct((2, D, B), jnp.float32),
        mesh=mesh,
        scratch_types=[
            pltpu.VMEM((2, S), jnp.int32),
            pltpu.VMEM((2, S, D), jnp.float32),
            pltpu.VMEM((RW,), jnp.int32),
            pltpu.VMEM((RW * D,), jnp.int32),
            pltpu.VMEM((RW * D,), jnp.float32),
            pltpu.VMEM((C * D,), jnp.float32),
            pltpu.VMEM((D, RW), jnp.float32),
            pltpu.VMEM((D,), jnp.float32),
            pltpu.SemaphoreType.DMA,
            pltpu.SemaphoreType.DMA,
        ],
        compiler_params=pltpu.CompilerParams(
            needs_layout_passes=False, use_tc_tiling_on_sc=False),
    )
    out_t = run(id_sparse, id_flat, wu_flat, W_hist)
    return jnp.transpose(out_t, (2, 0, 1))


# speedup vs baseline: 2.9503x; 2.9503x over previous
import jax
import jax.numpy as jnp
from jax import lax
from jax.experimental import pallas as pl
from jax.experimental.pallas import tpu as pltpu
from jax.experimental.pallas import tpu_sc as plsc

B, L, V, D = 16384, 50, 1000000, 32
NC, NS = 2, 16
NW = NC * NS
RW = B // NW
C = 16
NCHUNK = RW // C
S = C * L


def _body(id_sp_hbm, id_flat_hbm, wu_hbm, wh_hbm, out_hbm,
          ids_v, rows_v, sp_idx_v, sp_rows_v, st_v, out_t_v, w0_v,
          sem, sem2):
    wid = lax.axis_index("s") * NC + lax.axis_index("c")
    base = wid * RW

    pltpu.sync_copy(wh_hbm.at[0], w0_v)
    pltpu.sync_copy(id_sp_hbm.at[pl.ds(base, RW)], sp_idx_v)
    pltpu.async_copy(wu_hbm.at[sp_idx_v], sp_rows_v, sem2)

    pltpu.sync_copy(id_flat_hbm.at[pl.ds(base * L, S)], ids_v.at[0])
    pltpu.async_copy(wh_hbm.at[ids_v.at[0]], rows_v.at[0], sem)

    pltpu.make_async_copy(wu_hbm.at[sp_idx_v], sp_rows_v, sem2).wait()

    w00 = w0_v[pl.ds(0, 16)]
    w01 = w0_v[pl.ds(16, 16)]
    lane = lax.iota(jnp.int32, 16)
    pat32 = lane * 32

    def chunk(g, carry):
        p = lax.rem(g, 2)
        q = 1 - p
        row0 = base + g * C

        @pl.when(g < NCHUNK - 1)
        def _():
            pltpu.sync_copy(
                id_flat_hbm.at[pl.ds((row0 + C) * L, S)], ids_v.at[q])
            pltpu.async_copy(wh_hbm.at[ids_v.at[q]], rows_v.at[q], sem)

        pltpu.make_async_copy(wh_hbm.at[ids_v.at[p]], rows_v.at[p], sem).wait()

        for i in range(C):
            o = i * L
            e0 = ids_v[p, pl.ds(o, 16)]
            e1 = ids_v[p, pl.ds(o + 16, 16)]
            e2 = ids_v[p, pl.ds(o + 32, 16)]
            e3 = ids_v[p, pl.ds(o + 34, 16)]
            zc = (plsc.all_reduce_population_count(e0 == 0)
                  + plsc.all_reduce_population_count(e1 == 0)
                  + plsc.all_reduce_population_count(e2 == 0)
                  + plsc.all_reduce_population_count((e3 == 0) & (lane >= 14))
                  ).astype(jnp.float32)

            a = [rows_v[p, o + k, pl.ds(0, 16)] for k in range(4)]
            b = [rows_v[p, o + k, pl.ds(16, 16)] for k in range(4)]
            for l in range(4, L - 2, 4):
                for k in range(4):
                    a[k] = a[k] + rows_v[p, o + l + k, pl.ds(0, 16)]
                    b[k] = b[k] + rows_v[p, o + l + k, pl.ds(16, 16)]
            a[0] = a[0] + rows_v[p, o + L - 2, pl.ds(0, 16)]
            b[0] = b[0] + rows_v[p, o + L - 2, pl.ds(16, 16)]
            a[1] = a[1] + rows_v[p, o + L - 1, pl.ds(0, 16)]
            b[1] = b[1] + rows_v[p, o + L - 1, pl.ds(16, 16)]
            a0 = (a[0] + a[1]) + (a[2] + a[3])
            a1 = (b[0] + b[1]) + (b[2] + b[3])

            inv = 1.0 / (jnp.float32(L) - zc + 1e-8)
            st_v[pl.ds(i * 32, 16)] = (a0 - zc * w00) * inv
            st_v[pl.ds(i * 32 + 16, 16)] = (a1 - zc * w01) * inv

        bidx = g * C + lane
        for j in range(D):
            out_t_v[0, j, pl.ds(g * C, C)] = plsc.load_gather(
                sp_rows_v, [bidx, jnp.full((16,), j, jnp.int32)])
            out_t_v[1, j, pl.ds(g * C, C)] = plsc.load_gather(
                st_v, [pat32 + j])
        return carry

    lax.fori_loop(0, NCHUNK, chunk, 0)
    pltpu.sync_copy(out_t_v, out_hbm.at[:, :, pl.ds(base, RW)])


@jax.jit
def kernel(id_sparse, id_seq, W_user, W_hist):
    id_flat = id_seq.reshape(B * L)
    mesh = plsc.VectorSubcoreMesh(core_axis_name="c", subcore_axis_name="s")
    run = pl.kernel(
        _body,
        out_type=jax.ShapeDtypeStruct((2, D, B), jnp.float32),
        mesh=mesh,
        scratch_types=[
            pltpu.VMEM((2, S), jnp.int32),
            pltpu.VMEM((2, S, D), jnp.float32),
            pltpu.VMEM((RW,), jnp.int32),
            pltpu.VMEM((RW, D), jnp.float32),
            pltpu.VMEM((C * D,), jnp.float32),
            pltpu.VMEM((2, D, RW), jnp.float32),
            pltpu.VMEM((D,), jnp.float32),
            pltpu.SemaphoreType.DMA,
            pltpu.SemaphoreType.DMA,
        ],
        compiler_params=pltpu.CompilerParams(
            needs_layout_passes=False, use_tc_tiling_on_sc=False),
    )
    out_t = run(id_sparse, id_flat, W_user, W_hist)
    return jnp.transpose(out_t, (2, 0, 1))
